# baseline (speedup 1.0000x reference)
import jax
import jax.numpy as jnp
from jax.experimental import pallas as pl
from jax.experimental.pallas import tpu as pltpu

_HI = jax.lax.Precision.HIGHEST


def _rvq_body(x_ref, Wi_ref, bi_ref, Wo_ref, bo_ref, cbT_ref, cbM_ref,
              cbn_ref, out_ref, codes_ref):
    NQ, D, K = cbT_ref.shape
    xb = x_ref[0]
    z = jnp.dot(Wi_ref[...], xb,
                preferred_element_type=jnp.float32) + bi_ref[...]
    res = z
    TB = z.shape[1]
    idx_rows = []
    for i in range(NQ):
        cbT_i = cbT_ref[i]
        rn = jnp.sum(res * res, axis=0, keepdims=True)
        m2s = jax.lax.dot_general(
            cbM_ref[i], res, (((0,), (0,)), ((), ())),
            preferred_element_type=jnp.float32)
        dist = (rn + m2s) + cbn_ref[i]
        idx = jnp.argmin(dist, axis=0)[None, :]
        lo = jnp.broadcast_to(idx & 127, (D, TB))
        hi = jnp.broadcast_to(idx >> 7, (D, TB))
        q = None
        for c in range(K // 128):
            qc = jnp.take_along_axis(
                cbT_i[:, c * 128:(c + 1) * 128], lo, axis=1)
            q = qc if q is None else jnp.where(hi == c, qc, q)
        res = res - q
        idx_rows.append(idx)
    out_ref[0] = jnp.dot(Wo_ref[...], z - res,
                         preferred_element_type=jnp.float32) + bo_ref[...]
    codes_ref[0] = jnp.concatenate(idx_rows, axis=0)


def kernel(x, W_in, b_in, W_out, b_out, codebooks):
    B, C_IN, T = x.shape
    NQ, K, D = codebooks.shape
    C_OUT = W_out.shape[0]
    TB = 1024

    cbT = codebooks.transpose(0, 2, 1)
    cbn = jnp.sum(codebooks * codebooks, axis=-1)[..., None]
    cbM = -2.0 * cbT
    bi = b_in.reshape(D, 1)
    bo = b_out.reshape(C_OUT, 1)

    out, codes3 = pl.pallas_call(
        _rvq_body,
        grid=(B, T // TB),
        in_specs=[
            pl.BlockSpec((1, C_IN, TB), lambda b, t: (b, 0, t)),
            pl.BlockSpec((D, C_IN), lambda b, t: (0, 0)),
            pl.BlockSpec((D, 1), lambda b, t: (0, 0)),
            pl.BlockSpec((C_OUT, D), lambda b, t: (0, 0)),
            pl.BlockSpec((C_OUT, 1), lambda b, t: (0, 0)),
            pl.BlockSpec((NQ, D, K), lambda b, t: (0, 0, 0)),
            pl.BlockSpec((NQ, D, K), lambda b, t: (0, 0, 0)),
            pl.BlockSpec((NQ, K, 1), lambda b, t: (0, 0, 0)),
        ],
        out_specs=[
            pl.BlockSpec((1, C_OUT, TB), lambda b, t: (b, 0, t)),
            pl.BlockSpec((1, NQ, TB), lambda b, t: (b, 0, t)),
        ],
        out_shape=[
            jax.ShapeDtypeStruct((B, C_OUT, T), jnp.float32),
            jax.ShapeDtypeStruct((B, NQ, T), jnp.int32),
        ],
        compiler_params=pltpu.CompilerParams(
            dimension_semantics=("parallel", "parallel")),
    )(x, W_in, bi, W_out, bo, cbT, cbM, cbn)

    codes = codes3.transpose(1, 0, 2).reshape(NQ, B * T)
    return out, codes


# speedup vs baseline: 1.5738x; 1.5738x over previous
import jax
import jax.numpy as jnp
from jax.experimental import pallas as pl
from jax.experimental.pallas import tpu as pltpu

_HI = jax.lax.Precision.HIGHEST


def _rvq_body(x_ref, Wi_ref, bi_ref, Wo_ref, bo_ref, cbT_ref, cbA_ref,
              out_ref, codes_ref):
    NQ, D, K = cbT_ref.shape
    xb = x_ref[0]
    z = jnp.dot(Wi_ref[...], xb,
                preferred_element_type=jnp.float32) + bi_ref[...]
    res = z
    TB = z.shape[1]
    ones = jnp.ones((3, TB), jnp.float32)
    idx_rows = []
    for i in range(NQ):
        cbT_i = cbT_ref[i]
        resA = jnp.concatenate([res, ones], axis=0)
        dist = jax.lax.dot_general(
            cbA_ref[i], resA, (((0,), (0,)), ((), ())),
            preferred_element_type=jnp.float32)
        idx = jnp.argmin(dist, axis=0)[None, :]
        lo = jnp.broadcast_to(idx & 127, (D, TB))
        hi = jnp.broadcast_to(idx >> 7, (D, TB))
        q = None
        for c in range(K // 128):
            qc = jnp.take_along_axis(
                cbT_i[:, c * 128:(c + 1) * 128], lo, axis=1)
            q = qc if q is None else jnp.where(hi == c, qc, q)
        res = res - q
        idx_rows.append(idx)
    out_ref[0] = jnp.dot(Wo_ref[...], z - res,
                         preferred_element_type=jnp.float32) + bo_ref[...]
    codes_ref[0] = jnp.concatenate(idx_rows, axis=0)


def kernel(x, W_in, b_in, W_out, b_out, codebooks):
    B, C_IN, T = x.shape
    NQ, K, D = codebooks.shape
    C_OUT = W_out.shape[0]
    TB = 1024

    cbT = codebooks.transpose(0, 2, 1)
    cbn = jnp.sum(codebooks * codebooks, axis=-1)
    c0 = cbn.astype(jnp.bfloat16).astype(jnp.float32)
    r1 = cbn - c0
    c1 = r1.astype(jnp.bfloat16).astype(jnp.float32)
    c2 = (r1 - c1).astype(jnp.bfloat16).astype(jnp.float32)
    cbA = jnp.concatenate(
        [-2.0 * cbT, c0[:, None, :], c1[:, None, :], c2[:, None, :]],
        axis=1)
    bi = b_in.reshape(D, 1)
    bo = b_out.reshape(C_OUT, 1)

    out, codes3 = pl.pallas_call(
        _rvq_body,
        grid=(B, T // TB),
        in_specs=[
            pl.BlockSpec((1, C_IN, TB), lambda b, t: (b, 0, t)),
            pl.BlockSpec((D, C_IN), lambda b, t: (0, 0)),
            pl.BlockSpec((D, 1), lambda b, t: (0, 0)),
            pl.BlockSpec((C_OUT, D), lambda b, t: (0, 0)),
            pl.BlockSpec((C_OUT, 1), lambda b, t: (0, 0)),
            pl.BlockSpec((NQ, D, K), lambda b, t: (0, 0, 0)),
            pl.BlockSpec((NQ, D + 3, K), lambda b, t: (0, 0, 0)),
        ],
        out_specs=[
            pl.BlockSpec((1, C_OUT, TB), lambda b, t: (b, 0, t)),
            pl.BlockSpec((1, NQ, TB), lambda b, t: (b, 0, t)),
        ],
        out_shape=[
            jax.ShapeDtypeStruct((B, C_OUT, T), jnp.float32),
            jax.ShapeDtypeStruct((B, NQ, T), jnp.int32),
        ],
        compiler_params=pltpu.CompilerParams(
            dimension_semantics=("parallel", "parallel")),
    )(x, W_in, bi, W_out, bo, cbT, cbA)

    codes = codes3.transpose(1, 0, 2).reshape(NQ, B * T)
    return out, codes
